# baseline (speedup 1.0000x reference)
import functools

import jax
import jax.numpy as jnp
from jax import lax
from jax.experimental import pallas as pl
from jax.experimental.pallas import tpu as pltpu
from jax.experimental.pallas import tpu_sc as plsc

N = 10000
E = 320000
D = 128
NC, NS = 2, 16
NW = NC * NS
NPAD = 10240
RPT = NPAD // NS
CH = 128
EPW = 10240
K = EPW // CH
EPAD = NW * EPW
DW = 16
BLK = 320
GRID = NPAD // BLK

_mesh = plsc.VectorSubcoreMesh(core_axis_name="c", subcore_axis_name="s")


@functools.partial(
    pl.kernel,
    out_type=jax.ShapeDtypeStruct((NC * NPAD, D), jnp.float32),
    mesh=_mesh,
    scratch_types=[
        pltpu.VMEM((K, CH), jnp.int32),
        pltpu.VMEM((CH, D), jnp.float32),
        pltpu.VMEM_SHARED((NPAD, D), jnp.float32),
    ],
)
def _sc_degree(dst_hbm, ones_hbm, zeros_hbm, out_hbm, didx, ones_v, acc):
    cid = lax.axis_index("c")
    sid = lax.axis_index("s")
    wid = cid * NS + sid
    r0 = sid * RPT
    for z in range(RPT // CH):
        pltpu.sync_copy(zeros_hbm, acc.at[pl.ds(r0 + z * CH, CH)])
    pltpu.sync_copy(dst_hbm.at[wid], didx)
    pltpu.sync_copy(ones_hbm, ones_v)
    plsc.subcore_barrier()

    def body(j, c):
        pltpu.sync_copy(ones_v, acc.at[didx.at[j]], add=True)
        return c

    lax.fori_loop(0, K, body, 0)
    plsc.subcore_barrier()
    pltpu.sync_copy(acc.at[pl.ds(r0, RPT)], out_hbm.at[pl.ds(cid * NPAD + r0, RPT)])


K0 = 120
K1 = 40
KMAX = 120
SB = 8


@functools.partial(
    pl.kernel,
    out_type=jax.ShapeDtypeStruct((NC * NPAD, D), jnp.float32),
    mesh=_mesh,
    scratch_types=[
        pltpu.VMEM((KMAX, CH), jnp.int32),
        pltpu.VMEM((KMAX, CH), jnp.int32),
        pltpu.VMEM((CH, D), jnp.float32),
        pltpu.VMEM_SHARED((NPAD, D), jnp.float32),
        pltpu.SemaphoreType.DMA,
    ],
)
def _sc_aggregate(hs_hbm, src_hbm, dst_hbm, zeros_hbm, out_hbm,
                  sidx, didx, rows, acc, sem):
    cid = lax.axis_index("c")
    sid = lax.axis_index("s")
    wid = cid * NS + sid
    r0 = sid * RPT
    for z in range(RPT // CH):
        pltpu.sync_copy(zeros_hbm, acc.at[pl.ds(r0 + z * CH, CH)])
    pltpu.sync_copy(src_hbm.at[wid], sidx)
    pltpu.sync_copy(dst_hbm.at[wid], didx)
    plsc.subcore_barrier()

    def body(j, c):
        pltpu.async_copy(hs_hbm.at[sidx.at[j]], rows, sem).wait()
        pltpu.sync_copy(rows, acc.at[didx.at[j]], add=True)
        return c

    kc = jnp.where(cid == 0, K0, K1)
    lax.fori_loop(0, kc, body, 0)
    plsc.subcore_barrier()
    pltpu.sync_copy(acc.at[pl.ds(r0, RPT)], out_hbm.at[pl.ds(cid * NPAD + r0, RPT)])


@functools.partial(
    pl.kernel,
    out_type=jax.ShapeDtypeStruct((NC * NPAD, D), jnp.float32),
    mesh=_mesh,
    scratch_types=[
        pltpu.VMEM((SB, CH), jnp.int32),
        pltpu.VMEM((SB, CH), jnp.int32),
        pltpu.VMEM((CH, D), jnp.float32),
        pltpu.VMEM((CH, D), jnp.float32),
        pltpu.VMEM_SHARED((NPAD, D), jnp.float32),
        pltpu.SemaphoreType.DMA,
        pltpu.SemaphoreType.DMA,
    ],
)
def _sc_aggregate2(hs_hbm, src_hbm, dst_hbm, zeros_hbm, out_hbm,
                   sidxs, didxs, rows0, rows1, acc, sem0, sem1):
    cid = lax.axis_index("c")
    sid = lax.axis_index("s")
    wid = cid * NS + sid
    r0 = sid * RPT
    for z in range(RPT // CH):
        pltpu.sync_copy(zeros_hbm, acc.at[pl.ds(r0 + z * CH, CH)])
    plsc.subcore_barrier()

    def body(s, c):
        j0 = s * SB
        pltpu.sync_copy(src_hbm.at[pl.ds(wid * KMAX + j0, SB)], sidxs)
        pltpu.sync_copy(dst_hbm.at[pl.ds(wid * KMAX + j0, SB)], didxs)
        for u in range(SB // 2):
            h0 = pltpu.async_copy(hs_hbm.at[sidxs.at[2 * u]], rows0, sem0)
            h1 = pltpu.async_copy(hs_hbm.at[sidxs.at[2 * u + 1]], rows1, sem1)
            h0.wait()
            pltpu.sync_copy(rows0, acc.at[didxs.at[2 * u]], add=True)
            h1.wait()
            pltpu.sync_copy(rows1, acc.at[didxs.at[2 * u + 1]], add=True)
        return c

    kc = jnp.where(cid == 0, K0, K1)
    lax.fori_loop(0, kc // SB, body, 0)
    plsc.subcore_barrier()
    pltpu.sync_copy(acc.at[pl.ds(r0, RPT)], out_hbm.at[pl.ds(cid * NPAD + r0, RPT)])


def _dinv_of(degp_ref):
    dp = degp_ref[...]
    deg = dp[:, 0:1] + dp[:, 1:2] + 1.0
    return lax.rsqrt(deg)


def _tc1_body(x_ref, degp_ref, w_ref, o_ref):
    dinv = _dinv_of(degp_ref)
    h = jnp.dot(x_ref[...], w_ref[...], preferred_element_type=jnp.float32)
    o_ref[...] = h * dinv


def _tc_mid_body(agg_ref, hs_ref, degp_ref, b_ref, w_ref, o_ref):
    dinv = _dinv_of(degp_ref)
    a = agg_ref[...]
    t = jnp.maximum((a[0] + a[1] + hs_ref[...]) * dinv + b_ref[...], 0.0)
    h = jnp.dot(t, w_ref[...], preferred_element_type=jnp.float32)
    o_ref[...] = h * dinv


def _tc_out_body(agg_ref, hs_ref, degp_ref, b_ref, wt_ref, bout_ref, o_ref):
    dinv = _dinv_of(degp_ref)
    a = agg_ref[...]
    u = jnp.maximum((a[0] + a[1] + hs_ref[...]) * dinv + b_ref[...], 0.0)
    r = jnp.sum(u * wt_ref[...], axis=1, keepdims=True) + bout_ref[...]
    o_ref[...] = jnp.broadcast_to(r, (BLK, D))


_tc1 = pl.pallas_call(
    _tc1_body,
    grid=(GRID,),
    in_specs=[
        pl.BlockSpec((BLK, D), lambda i: (i, 0)),
        pl.BlockSpec((BLK, 2), lambda i: (i, 0)),
        pl.BlockSpec((D, D), lambda i: (0, 0)),
    ],
    out_specs=pl.BlockSpec((BLK, D), lambda i: (i, 0)),
    out_shape=jax.ShapeDtypeStruct((NPAD, D), jnp.float32),
)

_tc_mid = pl.pallas_call(
    _tc_mid_body,
    grid=(GRID,),
    in_specs=[
        pl.BlockSpec((2, BLK, D), lambda i: (0, i, 0)),
        pl.BlockSpec((BLK, D), lambda i: (i, 0)),
        pl.BlockSpec((BLK, 2), lambda i: (i, 0)),
        pl.BlockSpec((1, D), lambda i: (0, 0)),
        pl.BlockSpec((D, D), lambda i: (0, 0)),
    ],
    out_specs=pl.BlockSpec((BLK, D), lambda i: (i, 0)),
    out_shape=jax.ShapeDtypeStruct((NPAD, D), jnp.float32),
)

_tc_out = pl.pallas_call(
    _tc_out_body,
    grid=(GRID,),
    in_specs=[
        pl.BlockSpec((2, BLK, D), lambda i: (0, i, 0)),
        pl.BlockSpec((BLK, D), lambda i: (i, 0)),
        pl.BlockSpec((BLK, 2), lambda i: (i, 0)),
        pl.BlockSpec((1, D), lambda i: (0, 0)),
        pl.BlockSpec((1, D), lambda i: (0, 0)),
        pl.BlockSpec((1, 1), lambda i: (0, 0)),
    ],
    out_specs=pl.BlockSpec((BLK, D), lambda i: (i, 0)),
    out_shape=jax.ShapeDtypeStruct((NPAD, D), jnp.float32),
)


def kernel(x, edge_index, W1, b1, W2, b2, Wout, bout):
    si = edge_index[0].astype(jnp.int32)
    di = edge_index[1].astype(jnp.int32)
    fill = jnp.full((EPAD - E,), N, jnp.int32)

    def _asym(e):
        e0 = K0 * NS * CH
        p0 = e[:e0].reshape(NS, K0, CH)
        p1 = e[e0:].reshape(NS, K1, CH)

        def _padk(p, k):
            if k == KMAX:
                return p
            return jnp.concatenate(
                [p, jnp.full((NS, KMAX - k, CH), N, jnp.int32)], axis=1)

        return jnp.concatenate([_padk(p0, K0), _padk(p1, K1)], axis=0)

    src_all = jnp.concatenate([si, fill])
    dst_all = jnp.concatenate([di, fill])
    srcA = _asym(src_all)
    dstA = _asym(dst_all)
    src2 = srcA.reshape(NW * KMAX, CH)
    dst2 = dstA.reshape(NW * KMAX, CH)
    dst3 = dst_all.reshape(NW, K, CH)
    zer_a = jnp.zeros((CH, D), jnp.float32)
    ones_a = jnp.ones((CH, D), jnp.float32)

    degw = _sc_degree(dst3, ones_a, zer_a)
    degp = degw[:, 0].reshape(NC, NPAD).T

    xp = jnp.pad(x, ((0, NPAD - N), (0, 0)))
    hs1 = _tc1(xp, degp, W1)
    agg1 = _sc_aggregate2(hs1, src2, dst2, zer_a).reshape(NC, NPAD, D)
    hs2 = _tc_mid(agg1, hs1, degp, b1.reshape(1, D), W2)
    agg2 = _sc_aggregate2(hs2, src2, dst2, zer_a).reshape(NC, NPAD, D)
    ob = _tc_out(agg2, hs2, degp, b2.reshape(1, D),
                 Wout.reshape(1, D), bout.reshape(1, 1))
    return ob[:N, 0]


# speedup vs baseline: 1.0317x; 1.0317x over previous
import functools

import jax
import jax.numpy as jnp
from jax import lax
from jax.experimental import pallas as pl
from jax.experimental.pallas import tpu as pltpu
from jax.experimental.pallas import tpu_sc as plsc

N = 10000
E = 320000
D = 128
NC, NS = 2, 16
NW = NC * NS
NPAD = 10240
RPT = NPAD // NS
CH = 128
EPW = 10240
K = EPW // CH
EPAD = NW * EPW
DW = 16
BLK = 320
GRID = NPAD // BLK

_mesh = plsc.VectorSubcoreMesh(core_axis_name="c", subcore_axis_name="s")


@functools.partial(
    pl.kernel,
    out_type=jax.ShapeDtypeStruct((NC * NPAD, D), jnp.float32),
    mesh=_mesh,
    scratch_types=[
        pltpu.VMEM((K, CH), jnp.int32),
        pltpu.VMEM((CH, D), jnp.float32),
        pltpu.VMEM_SHARED((NPAD, D), jnp.float32),
    ],
)
def _sc_degree(dst_hbm, ones_hbm, zeros_hbm, out_hbm, didx, ones_v, acc):
    cid = lax.axis_index("c")
    sid = lax.axis_index("s")
    wid = cid * NS + sid
    r0 = sid * RPT
    for z in range(RPT // CH):
        pltpu.sync_copy(zeros_hbm, acc.at[pl.ds(r0 + z * CH, CH)])
    pltpu.sync_copy(dst_hbm.at[wid], didx)
    pltpu.sync_copy(ones_hbm, ones_v)
    plsc.subcore_barrier()

    def body(j, c):
        pltpu.sync_copy(ones_v, acc.at[didx.at[j]], add=True)
        return c

    lax.fori_loop(0, K, body, 0)
    plsc.subcore_barrier()
    pltpu.sync_copy(acc.at[pl.ds(r0, RPT)], out_hbm.at[pl.ds(cid * NPAD + r0, RPT)])


K0 = 127
K1 = 33
KMAX = 127


@functools.partial(
    pl.kernel,
    out_type=jax.ShapeDtypeStruct((NC * NPAD, D), jnp.float32),
    mesh=_mesh,
    scratch_types=[
        pltpu.VMEM((KMAX, CH), jnp.int32),
        pltpu.VMEM((KMAX, CH), jnp.int32),
        pltpu.VMEM((CH, D), jnp.float32),
        pltpu.VMEM_SHARED((NPAD, D), jnp.float32),
        pltpu.SemaphoreType.DMA,
    ],
)
def _sc_aggregate(hs_hbm, src_hbm, dst_hbm, zeros_hbm, out_hbm,
                  sidx, didx, rows, acc, sem):
    cid = lax.axis_index("c")
    sid = lax.axis_index("s")
    wid = cid * NS + sid
    r0 = sid * RPT
    for z in range(RPT // CH):
        pltpu.sync_copy(zeros_hbm, acc.at[pl.ds(r0 + z * CH, CH)])
    pltpu.sync_copy(src_hbm.at[wid], sidx)
    pltpu.sync_copy(dst_hbm.at[wid], didx)
    plsc.subcore_barrier()

    def body(j, c):
        pltpu.async_copy(hs_hbm.at[sidx.at[j]], rows, sem).wait()
        pltpu.sync_copy(rows, acc.at[didx.at[j]], add=True)
        return c

    kc = jnp.where(cid == 0, K0, K1)
    lax.fori_loop(0, kc, body, 0)
    plsc.subcore_barrier()
    pltpu.sync_copy(acc.at[pl.ds(r0, RPT)], out_hbm.at[pl.ds(cid * NPAD + r0, RPT)])


def _dinv_of(degp_ref):
    dp = degp_ref[...]
    deg = dp[:, 0:1] + dp[:, 1:2] + 1.0
    return lax.rsqrt(deg)


def _tc1_body(x_ref, degp_ref, w_ref, o_ref):
    dinv = _dinv_of(degp_ref)
    h = jnp.dot(x_ref[...], w_ref[...], preferred_element_type=jnp.float32)
    o_ref[...] = h * dinv


def _tc_mid_body(agg_ref, hs_ref, degp_ref, b_ref, w_ref, o_ref):
    dinv = _dinv_of(degp_ref)
    a = agg_ref[...]
    t = jnp.maximum((a[0] + a[1] + hs_ref[...]) * dinv + b_ref[...], 0.0)
    h = jnp.dot(t, w_ref[...], preferred_element_type=jnp.float32)
    o_ref[...] = h * dinv


def _tc_out_body(agg_ref, hs_ref, degp_ref, b_ref, wt_ref, bout_ref, o_ref):
    dinv = _dinv_of(degp_ref)
    a = agg_ref[...]
    u = jnp.maximum((a[0] + a[1] + hs_ref[...]) * dinv + b_ref[...], 0.0)
    r = jnp.sum(u * wt_ref[...], axis=1, keepdims=True) + bout_ref[...]
    o_ref[...] = jnp.broadcast_to(r, (BLK, D))


_tc1 = pl.pallas_call(
    _tc1_body,
    grid=(GRID,),
    in_specs=[
        pl.BlockSpec((BLK, D), lambda i: (i, 0)),
        pl.BlockSpec((BLK, 2), lambda i: (i, 0)),
        pl.BlockSpec((D, D), lambda i: (0, 0)),
    ],
    out_specs=pl.BlockSpec((BLK, D), lambda i: (i, 0)),
    out_shape=jax.ShapeDtypeStruct((NPAD, D), jnp.float32),
)

_tc_mid = pl.pallas_call(
    _tc_mid_body,
    grid=(GRID,),
    in_specs=[
        pl.BlockSpec((2, BLK, D), lambda i: (0, i, 0)),
        pl.BlockSpec((BLK, D), lambda i: (i, 0)),
        pl.BlockSpec((BLK, 2), lambda i: (i, 0)),
        pl.BlockSpec((1, D), lambda i: (0, 0)),
        pl.BlockSpec((D, D), lambda i: (0, 0)),
    ],
    out_specs=pl.BlockSpec((BLK, D), lambda i: (i, 0)),
    out_shape=jax.ShapeDtypeStruct((NPAD, D), jnp.float32),
)

_tc_out = pl.pallas_call(
    _tc_out_body,
    grid=(GRID,),
    in_specs=[
        pl.BlockSpec((2, BLK, D), lambda i: (0, i, 0)),
        pl.BlockSpec((BLK, D), lambda i: (i, 0)),
        pl.BlockSpec((BLK, 2), lambda i: (i, 0)),
        pl.BlockSpec((1, D), lambda i: (0, 0)),
        pl.BlockSpec((1, D), lambda i: (0, 0)),
        pl.BlockSpec((1, 1), lambda i: (0, 0)),
    ],
    out_specs=pl.BlockSpec((BLK, D), lambda i: (i, 0)),
    out_shape=jax.ShapeDtypeStruct((NPAD, D), jnp.float32),
)


def kernel(x, edge_index, W1, b1, W2, b2, Wout, bout):
    si = edge_index[0].astype(jnp.int32)
    di = edge_index[1].astype(jnp.int32)
    fill = jnp.full((EPAD - E,), N, jnp.int32)

    def _asym(e):
        e0 = K0 * NS * CH
        p0 = e[:e0].reshape(NS, K0, CH)
        p1 = e[e0:].reshape(NS, K1, CH)

        def _padk(p, k):
            if k == KMAX:
                return p
            return jnp.concatenate(
                [p, jnp.full((NS, KMAX - k, CH), N, jnp.int32)], axis=1)

        return jnp.concatenate([_padk(p0, K0), _padk(p1, K1)], axis=0)

    src_all = jnp.concatenate([si, fill])
    dst_all = jnp.concatenate([di, fill])
    srcA = _asym(src_all)
    dstA = _asym(dst_all)
    dst3 = dst_all.reshape(NW, K, CH)
    zer_a = jnp.zeros((CH, D), jnp.float32)
    ones_a = jnp.ones((CH, D), jnp.float32)

    degw = _sc_degree(dst3, ones_a, zer_a)
    degp = degw[:, 0].reshape(NC, NPAD).T

    xp = jnp.pad(x, ((0, NPAD - N), (0, 0)))
    hs1 = _tc1(xp, degp, W1)
    agg1 = _sc_aggregate(hs1, srcA, dstA, zer_a).reshape(NC, NPAD, D)
    hs2 = _tc_mid(agg1, hs1, degp, b1.reshape(1, D), W2)
    agg2 = _sc_aggregate(hs2, srcA, dstA, zer_a).reshape(NC, NPAD, D)
    ob = _tc_out(agg2, hs2, degp, b2.reshape(1, D),
                 Wout.reshape(1, D), bout.reshape(1, 1))
    return ob[:N, 0]
